# baseline (speedup 1.0000x reference)
import functools

import jax
import jax.numpy as jnp
from jax import lax
from jax.experimental import pallas as pl
from jax.experimental.pallas import tpu as pltpu
from jax.experimental.pallas import tpu_sc as plsc

B = 16384
V1 = 1000001
D = 32
DP = 128
H1 = 128
H2 = 64

_W = 1024
_TG = -(-V1 // (4 * _W))
_VP4 = _TG * _W

_NC = 2
_NS = 16
_NW = _NC * _NS
_BPW = B // _NW

_mesh = plsc.VectorSubcoreMesh(core_axis_name="c", subcore_axis_name="s")

def _tpad_body(t_ref, o_ref):
    x = t_ref[...]
    o_ref[...] = jnp.concatenate(
        [x[:, a * _W:(a + 1) * _W].T for a in range(4)], axis=1)


def _tpad(tt):
    return pl.pallas_call(
        _tpad_body,
        grid=(_TG,),
        in_specs=[pl.BlockSpec((D, 4 * _W), lambda c: (0, c))],
        out_specs=pl.BlockSpec((_W, DP), lambda c: (c, 0)),
        out_shape=jax.ShapeDtypeStruct((_VP4, DP), jnp.float32),
    )(tt)


@functools.partial(
    pl.kernel,
    mesh=_mesh,
    out_type=jax.ShapeDtypeStruct((B, DP), jnp.float32),
    scratch_types=[
        pltpu.VMEM((_BPW,), jnp.int32),
        pltpu.VMEM((_BPW, DP), jnp.float32),
        pltpu.SemaphoreType.DMA,
    ],
    compiler_params=pltpu.CompilerParams(use_tc_tiling_on_sc=True),
)
def _sc_gather(idx_hbm, tab_hbm, out_hbm, idx_v, rows_v, sem):
    wid = lax.axis_index("s") * _NC + lax.axis_index("c")
    base = wid * _BPW
    pltpu.sync_copy(idx_hbm.at[pl.ds(base, _BPW)], idx_v)
    pltpu.async_copy(tab_hbm.at[idx_v], rows_v, sem).wait()
    pltpu.sync_copy(rows_v, out_hbm.at[pl.ds(base, _BPW)])


def _towers_body(xu_ref, au_ref, xi_ref, ai_ref,
                 uW1_ref, ub1_ref, uW2_ref, ub2_ref, uW3_ref, ub3_ref,
                 iW1_ref, ib1_ref, iW2_ref, ib2_ref, iW3_ref, ib3_ref,
                 uo_ref, io_ref):
    def select(x128, a):
        x = jnp.where(a == 0, x128[:, 0:D], x128[:, D:2 * D])
        x = jnp.where(a == 2, x128[:, 2 * D:3 * D], x)
        return jnp.where(a == 3, x128[:, 3 * D:4 * D], x)

    def tower(x, W1, b1, W2, b2, W3, b3):
        h = jnp.maximum(jnp.dot(x, W1, preferred_element_type=jnp.float32) + b1, 0.0)
        h = jnp.maximum(jnp.dot(h, W2, preferred_element_type=jnp.float32) + b2, 0.0)
        return jnp.dot(h, W3, preferred_element_type=jnp.float32) + b3

    xu = select(xu_ref[...], au_ref[...])
    xi = select(xi_ref[...], ai_ref[...])
    uo_ref[...] = tower(xu, uW1_ref[...], ub1_ref[...], uW2_ref[...],
                        ub2_ref[...], uW3_ref[...], ub3_ref[...])
    io_ref[...] = tower(xi, iW1_ref[...], ib1_ref[...], iW2_ref[...],
                        ib2_ref[...], iW3_ref[...], ib3_ref[...])


_BLK = 2048


def _towers(xu, au, xi, ai, weights):
    row_spec = pl.BlockSpec((_BLK, DP), lambda i: (i, 0))
    a_spec = pl.BlockSpec((_BLK, 1), lambda i: (i, 0))
    out_spec = pl.BlockSpec((_BLK, D), lambda i: (i, 0))
    full = lambda shape: pl.BlockSpec(shape, lambda i: (0,) * len(shape))
    w_specs = [
        full((D, H1)), full((1, H1)), full((H1, H2)), full((1, H2)),
        full((H2, D)), full((1, D)),
        full((D, H1)), full((1, H1)), full((H1, H2)), full((1, H2)),
        full((H2, D)), full((1, D)),
    ]
    return pl.pallas_call(
        _towers_body,
        grid=(B // _BLK,),
        in_specs=[row_spec, a_spec, row_spec, a_spec] + w_specs,
        out_specs=[out_spec, out_spec],
        out_shape=[
            jax.ShapeDtypeStruct((B, D), jnp.float32),
            jax.ShapeDtypeStruct((B, D), jnp.float32),
        ],
    )(xu, au, xi, ai, *weights)


def kernel(user_id, movie_id, user_table, item_table,
           uW1, ub1, uW2, ub2, uW3, ub3,
           iW1, ib1, iW2, ib2, iW3, ib3):
    uid = user_id.astype(jnp.int32)
    mid = movie_id.astype(jnp.int32)
    utab = _tpad(user_table.T)
    itab = _tpad(item_table.T)

    def packed_row(v):
        return jnp.bitwise_or(jnp.left_shift(jnp.right_shift(v, 12), 10),
                              jnp.bitwise_and(v, _W - 1))

    xu = _sc_gather(packed_row(uid), utab)
    xi = _sc_gather(packed_row(mid), itab)
    au = jnp.bitwise_and(jnp.right_shift(uid, 10), 3).reshape(B, 1)
    ai = jnp.bitwise_and(jnp.right_shift(mid, 10), 3).reshape(B, 1)
    weights = (uW1, ub1.reshape(1, H1), uW2, ub2.reshape(1, H2), uW3,
               ub3.reshape(1, D),
               iW1, ib1.reshape(1, H1), iW2, ib2.reshape(1, H2), iW3,
               ib3.reshape(1, D))
    return _towers(xu, au, xi, ai, weights)


# speedup vs baseline: 1.1515x; 1.1515x over previous
import functools

import jax
import jax.numpy as jnp
from jax import lax
from jax.experimental import pallas as pl
from jax.experimental.pallas import tpu as pltpu
from jax.experimental.pallas import tpu_sc as plsc

B = 16384
V1 = 1000001
D = 32
DP = 128
H1 = 128
H2 = 64

_W = 2048
_LW = 11
_TG = -(-V1 // (4 * _W))
_VP4 = _TG * _W

_NC = 2
_NS = 16
_NW = _NC * _NS
_BPW = B // _NW

_mesh = plsc.VectorSubcoreMesh(core_axis_name="c", subcore_axis_name="s")

def _tpad_body(t_ref, o_ref):
    x = t_ref[...]
    eye = jnp.eye(D, dtype=jnp.float32)
    cdim = (((0,), (0,)), ((), ()))

    def tr(xc):
        return lax.dot_general(xc, eye, cdim,
                               preferred_element_type=jnp.float32)

    o_ref[...] = jnp.concatenate(
        [tr(x[:, a * _W:(a + 1) * _W]) for a in range(4)], axis=1)


def _tpad(tt):
    return pl.pallas_call(
        _tpad_body,
        grid=(_TG,),
        in_specs=[pl.BlockSpec((D, 4 * _W), lambda c: (0, c))],
        out_specs=pl.BlockSpec((_W, DP), lambda c: (c, 0)),
        out_shape=jax.ShapeDtypeStruct((_VP4, DP), jnp.float32),
    )(tt)


@functools.partial(
    pl.kernel,
    mesh=_mesh,
    out_type=jax.ShapeDtypeStruct((B, DP), jnp.float32),
    scratch_types=[
        pltpu.VMEM((_BPW,), jnp.int32),
        pltpu.VMEM((_BPW, DP), jnp.float32),
        pltpu.SemaphoreType.DMA,
    ],
    compiler_params=pltpu.CompilerParams(use_tc_tiling_on_sc=True),
)
def _sc_gather(idx_hbm, tab_hbm, out_hbm, idx_v, rows_v, sem):
    wid = lax.axis_index("s") * _NC + lax.axis_index("c")
    base = wid * _BPW
    pltpu.sync_copy(idx_hbm.at[pl.ds(base, _BPW)], idx_v)
    pltpu.async_copy(tab_hbm.at[idx_v], rows_v, sem).wait()
    pltpu.sync_copy(rows_v, out_hbm.at[pl.ds(base, _BPW)])


def _towers_body(xu_ref, au_ref, xi_ref, ai_ref,
                 uW1_ref, ub1_ref, uW2_ref, ub2_ref, uW3_ref, ub3_ref,
                 iW1_ref, ib1_ref, iW2_ref, ib2_ref, iW3_ref, ib3_ref,
                 uo_ref, io_ref):
    def select(x128, a):
        x = jnp.where(a == 0, x128[:, 0:D], x128[:, D:2 * D])
        x = jnp.where(a == 2, x128[:, 2 * D:3 * D], x)
        return jnp.where(a == 3, x128[:, 3 * D:4 * D], x)

    def tower(x, W1, b1, W2, b2, W3, b3):
        h = jnp.maximum(jnp.dot(x, W1, preferred_element_type=jnp.float32) + b1, 0.0)
        h = jnp.maximum(jnp.dot(h, W2, preferred_element_type=jnp.float32) + b2, 0.0)
        return jnp.dot(h, W3, preferred_element_type=jnp.float32) + b3

    xu = select(xu_ref[...], au_ref[...])
    xi = select(xi_ref[...], ai_ref[...])
    uo_ref[...] = tower(xu, uW1_ref[...], ub1_ref[...], uW2_ref[...],
                        ub2_ref[...], uW3_ref[...], ub3_ref[...])
    io_ref[...] = tower(xi, iW1_ref[...], ib1_ref[...], iW2_ref[...],
                        ib2_ref[...], iW3_ref[...], ib3_ref[...])


_BLK = 2048


def _towers(xu, au, xi, ai, weights):
    row_spec = pl.BlockSpec((_BLK, DP), lambda i: (i, 0))
    a_spec = pl.BlockSpec((_BLK, 1), lambda i: (i, 0))
    out_spec = pl.BlockSpec((_BLK, D), lambda i: (i, 0))
    full = lambda shape: pl.BlockSpec(shape, lambda i: (0,) * len(shape))
    w_specs = [
        full((D, H1)), full((1, H1)), full((H1, H2)), full((1, H2)),
        full((H2, D)), full((1, D)),
        full((D, H1)), full((1, H1)), full((H1, H2)), full((1, H2)),
        full((H2, D)), full((1, D)),
    ]
    return pl.pallas_call(
        _towers_body,
        grid=(B // _BLK,),
        in_specs=[row_spec, a_spec, row_spec, a_spec] + w_specs,
        out_specs=[out_spec, out_spec],
        out_shape=[
            jax.ShapeDtypeStruct((B, D), jnp.float32),
            jax.ShapeDtypeStruct((B, D), jnp.float32),
        ],
    )(xu, au, xi, ai, *weights)


def kernel(user_id, movie_id, user_table, item_table,
           uW1, ub1, uW2, ub2, uW3, ub3,
           iW1, ib1, iW2, ib2, iW3, ib3):
    uid = user_id.astype(jnp.int32)
    mid = movie_id.astype(jnp.int32)
    utab = _tpad(user_table.T)
    itab = _tpad(item_table.T)

    def packed_row(v):
        return jnp.bitwise_or(
            jnp.left_shift(jnp.right_shift(v, _LW + 2), _LW),
            jnp.bitwise_and(v, _W - 1))

    xu = _sc_gather(packed_row(uid), utab)
    xi = _sc_gather(packed_row(mid), itab)
    au = jnp.bitwise_and(jnp.right_shift(uid, _LW), 3).reshape(B, 1)
    ai = jnp.bitwise_and(jnp.right_shift(mid, _LW), 3).reshape(B, 1)
    weights = (uW1, ub1.reshape(1, H1), uW2, ub2.reshape(1, H2), uW3,
               ub3.reshape(1, D),
               iW1, ib1.reshape(1, H1), iW2, ib2.reshape(1, H2), iW3,
               ib3.reshape(1, D))
    return _towers(xu, au, xi, ai, weights)


# speedup vs baseline: 1.1780x; 1.0230x over previous
import functools

import jax
import jax.numpy as jnp
from jax import lax
from jax.experimental import pallas as pl
from jax.experimental.pallas import tpu as pltpu
from jax.experimental.pallas import tpu_sc as plsc

B = 16384
V1 = 1000001
D = 32
DP = 128
H1 = 128
H2 = 64

_W = 8192
_LW = 13
_TG = -(-V1 // (4 * _W))
_VP4 = _TG * _W

_NC = 2
_NS = 16
_NW = _NC * _NS
_BPW = B // _NW

_mesh = plsc.VectorSubcoreMesh(core_axis_name="c", subcore_axis_name="s")

def _tpad_body(t_ref, o_ref):
    x = t_ref[...]
    eye = jnp.eye(D, dtype=jnp.float32)
    cdim = (((0,), (0,)), ((), ()))

    def tr(xc):
        return lax.dot_general(xc, eye, cdim,
                               preferred_element_type=jnp.float32)

    o_ref[...] = jnp.concatenate(
        [tr(x[:, a * _W:(a + 1) * _W]) for a in range(4)], axis=1)


def _tpad(tt):
    return pl.pallas_call(
        _tpad_body,
        grid=(_TG,),
        in_specs=[pl.BlockSpec((D, 4 * _W), lambda c: (0, c))],
        out_specs=pl.BlockSpec((_W, DP), lambda c: (c, 0)),
        out_shape=jax.ShapeDtypeStruct((_VP4, DP), jnp.float32),
    )(tt)


@functools.partial(
    pl.kernel,
    mesh=_mesh,
    out_type=jax.ShapeDtypeStruct((B, DP), jnp.float32),
    scratch_types=[
        pltpu.VMEM((_BPW,), jnp.int32),
        pltpu.VMEM((_BPW, DP), jnp.float32),
        pltpu.SemaphoreType.DMA,
    ],
    compiler_params=pltpu.CompilerParams(use_tc_tiling_on_sc=True),
)
def _sc_gather(idx_hbm, tab_hbm, out_hbm, idx_v, rows_v, sem):
    wid = lax.axis_index("s") * _NC + lax.axis_index("c")
    base = wid * _BPW
    pltpu.sync_copy(idx_hbm.at[pl.ds(base, _BPW)], idx_v)
    pltpu.async_copy(tab_hbm.at[idx_v], rows_v, sem).wait()
    pltpu.sync_copy(rows_v, out_hbm.at[pl.ds(base, _BPW)])


def _towers_body(xu_ref, au_ref, xi_ref, ai_ref,
                 uW1_ref, ub1_ref, uW2_ref, ub2_ref, uW3_ref, ub3_ref,
                 iW1_ref, ib1_ref, iW2_ref, ib2_ref, iW3_ref, ib3_ref,
                 uo_ref, io_ref):
    def select(x128, a):
        x = jnp.where(a == 0, x128[:, 0:D], x128[:, D:2 * D])
        x = jnp.where(a == 2, x128[:, 2 * D:3 * D], x)
        return jnp.where(a == 3, x128[:, 3 * D:4 * D], x)

    def tower(x, W1, b1, W2, b2, W3, b3):
        h = jnp.maximum(jnp.dot(x, W1, preferred_element_type=jnp.float32) + b1, 0.0)
        h = jnp.maximum(jnp.dot(h, W2, preferred_element_type=jnp.float32) + b2, 0.0)
        return jnp.dot(h, W3, preferred_element_type=jnp.float32) + b3

    xu = select(xu_ref[...], au_ref[...])
    xi = select(xi_ref[...], ai_ref[...])
    uo_ref[...] = tower(xu, uW1_ref[...], ub1_ref[...], uW2_ref[...],
                        ub2_ref[...], uW3_ref[...], ub3_ref[...])
    io_ref[...] = tower(xi, iW1_ref[...], ib1_ref[...], iW2_ref[...],
                        ib2_ref[...], iW3_ref[...], ib3_ref[...])


_BLK = 2048


def _towers(xu, au, xi, ai, weights):
    row_spec = pl.BlockSpec((_BLK, DP), lambda i: (i, 0))
    a_spec = pl.BlockSpec((_BLK, 1), lambda i: (i, 0))
    out_spec = pl.BlockSpec((_BLK, D), lambda i: (i, 0))
    full = lambda shape: pl.BlockSpec(shape, lambda i: (0,) * len(shape))
    w_specs = [
        full((D, H1)), full((1, H1)), full((H1, H2)), full((1, H2)),
        full((H2, D)), full((1, D)),
        full((D, H1)), full((1, H1)), full((H1, H2)), full((1, H2)),
        full((H2, D)), full((1, D)),
    ]
    return pl.pallas_call(
        _towers_body,
        grid=(B // _BLK,),
        in_specs=[row_spec, a_spec, row_spec, a_spec] + w_specs,
        out_specs=[out_spec, out_spec],
        out_shape=[
            jax.ShapeDtypeStruct((B, D), jnp.float32),
            jax.ShapeDtypeStruct((B, D), jnp.float32),
        ],
    )(xu, au, xi, ai, *weights)


def kernel(user_id, movie_id, user_table, item_table,
           uW1, ub1, uW2, ub2, uW3, ub3,
           iW1, ib1, iW2, ib2, iW3, ib3):
    uid = user_id.astype(jnp.int32)
    mid = movie_id.astype(jnp.int32)
    utab = _tpad(user_table.T)
    itab = _tpad(item_table.T)

    def packed_row(v):
        return jnp.bitwise_or(
            jnp.left_shift(jnp.right_shift(v, _LW + 2), _LW),
            jnp.bitwise_and(v, _W - 1))

    xu = _sc_gather(packed_row(uid), utab)
    xi = _sc_gather(packed_row(mid), itab)
    au = jnp.bitwise_and(jnp.right_shift(uid, _LW), 3).reshape(B, 1)
    ai = jnp.bitwise_and(jnp.right_shift(mid, _LW), 3).reshape(B, 1)
    weights = (uW1, ub1.reshape(1, H1), uW2, ub2.reshape(1, H2), uW3,
               ub3.reshape(1, D),
               iW1, ib1.reshape(1, H1), iW2, ib2.reshape(1, H2), iW3,
               ib3.reshape(1, D))
    return _towers(xu, au, xi, ai, weights)


# speedup vs baseline: 2.6192x; 2.2234x over previous
import functools

import jax
import jax.numpy as jnp
from jax import lax
from jax.experimental import pallas as pl
from jax.experimental.pallas import tpu as pltpu
from jax.experimental.pallas import tpu_sc as plsc

B = 16384
V1 = 1000001
D = 32
DP = 128
H1 = 128
H2 = 64

_W = 8192
_LW = 13
_TG = -(-V1 // (4 * _W))
_VP4 = _TG * _W

_NC = 2
_NS = 16
_NW = _NC * _NS
_BPW = B // _NW

_mesh = plsc.VectorSubcoreMesh(core_axis_name="c", subcore_axis_name="s")

def _tpad_body(t_ref, o_ref):
    x = t_ref[...]
    xs = jnp.concatenate([x[:, a * _W:(a + 1) * _W] for a in range(4)],
                         axis=0)
    eye = jnp.eye(DP, dtype=jnp.float32)
    o_ref[...] = lax.dot_general(xs, eye, (((0,), (0,)), ((), ())),
                                 preferred_element_type=jnp.float32)


def _tpad(tt):
    return pl.pallas_call(
        _tpad_body,
        grid=(_TG,),
        in_specs=[pl.BlockSpec((D, 4 * _W), lambda c: (0, c))],
        out_specs=pl.BlockSpec((_W, DP), lambda c: (c, 0)),
        out_shape=jax.ShapeDtypeStruct((_VP4, DP), jnp.float32),
    )(tt)


@functools.partial(
    pl.kernel,
    mesh=_mesh,
    out_type=jax.ShapeDtypeStruct((B, DP), jnp.float32),
    scratch_types=[
        pltpu.VMEM((_BPW,), jnp.int32),
        pltpu.VMEM((_BPW, DP), jnp.float32),
        pltpu.SemaphoreType.DMA,
    ],
    compiler_params=pltpu.CompilerParams(use_tc_tiling_on_sc=True),
)
def _sc_gather(idx_hbm, tab_hbm, out_hbm, idx_v, rows_v, sem):
    wid = lax.axis_index("s") * _NC + lax.axis_index("c")
    base = wid * _BPW
    pltpu.sync_copy(idx_hbm.at[pl.ds(base, _BPW)], idx_v)
    pltpu.async_copy(tab_hbm.at[idx_v], rows_v, sem).wait()
    pltpu.sync_copy(rows_v, out_hbm.at[pl.ds(base, _BPW)])


def _towers_body(xu_ref, au_ref, xi_ref, ai_ref,
                 uW1_ref, ub1_ref, uW2_ref, ub2_ref, uW3_ref, ub3_ref,
                 iW1_ref, ib1_ref, iW2_ref, ib2_ref, iW3_ref, ib3_ref,
                 uo_ref, io_ref):
    def select(x128, a):
        x = jnp.where(a == 0, x128[:, 0:D], x128[:, D:2 * D])
        x = jnp.where(a == 2, x128[:, 2 * D:3 * D], x)
        return jnp.where(a == 3, x128[:, 3 * D:4 * D], x)

    def tower(x, W1, b1, W2, b2, W3, b3):
        h = jnp.maximum(jnp.dot(x, W1, preferred_element_type=jnp.float32) + b1, 0.0)
        h = jnp.maximum(jnp.dot(h, W2, preferred_element_type=jnp.float32) + b2, 0.0)
        return jnp.dot(h, W3, preferred_element_type=jnp.float32) + b3

    xu = select(xu_ref[...], au_ref[...])
    xi = select(xi_ref[...], ai_ref[...])
    uo_ref[...] = tower(xu, uW1_ref[...], ub1_ref[...], uW2_ref[...],
                        ub2_ref[...], uW3_ref[...], ub3_ref[...])
    io_ref[...] = tower(xi, iW1_ref[...], ib1_ref[...], iW2_ref[...],
                        ib2_ref[...], iW3_ref[...], ib3_ref[...])


_BLK = 2048


def _towers(xu, au, xi, ai, weights):
    row_spec = pl.BlockSpec((_BLK, DP), lambda i: (i, 0))
    a_spec = pl.BlockSpec((_BLK, 1), lambda i: (i, 0))
    out_spec = pl.BlockSpec((_BLK, D), lambda i: (i, 0))
    full = lambda shape: pl.BlockSpec(shape, lambda i: (0,) * len(shape))
    w_specs = [
        full((D, H1)), full((1, H1)), full((H1, H2)), full((1, H2)),
        full((H2, D)), full((1, D)),
        full((D, H1)), full((1, H1)), full((H1, H2)), full((1, H2)),
        full((H2, D)), full((1, D)),
    ]
    return pl.pallas_call(
        _towers_body,
        grid=(B // _BLK,),
        in_specs=[row_spec, a_spec, row_spec, a_spec] + w_specs,
        out_specs=[out_spec, out_spec],
        out_shape=[
            jax.ShapeDtypeStruct((B, D), jnp.float32),
            jax.ShapeDtypeStruct((B, D), jnp.float32),
        ],
    )(xu, au, xi, ai, *weights)


def kernel(user_id, movie_id, user_table, item_table,
           uW1, ub1, uW2, ub2, uW3, ub3,
           iW1, ib1, iW2, ib2, iW3, ib3):
    uid = user_id.astype(jnp.int32)
    mid = movie_id.astype(jnp.int32)
    utab = _tpad(user_table.T)
    itab = _tpad(item_table.T)

    def packed_row(v):
        return jnp.bitwise_or(
            jnp.left_shift(jnp.right_shift(v, _LW + 2), _LW),
            jnp.bitwise_and(v, _W - 1))

    xu = _sc_gather(packed_row(uid), utab)
    xi = _sc_gather(packed_row(mid), itab)
    au = jnp.bitwise_and(jnp.right_shift(uid, _LW), 3).reshape(B, 1)
    ai = jnp.bitwise_and(jnp.right_shift(mid, _LW), 3).reshape(B, 1)
    weights = (uW1, ub1.reshape(1, H1), uW2, ub2.reshape(1, H2), uW3,
               ub3.reshape(1, D),
               iW1, ib1.reshape(1, H1), iW2, ib2.reshape(1, H2), iW3,
               ib3.reshape(1, D))
    return _towers(xu, au, xi, ai, weights)


# speedup vs baseline: 2.8146x; 1.0746x over previous
import functools

import jax
import jax.numpy as jnp
from jax import lax
from jax.experimental import pallas as pl
from jax.experimental.pallas import tpu as pltpu
from jax.experimental.pallas import tpu_sc as plsc

B = 16384
V1 = 1000001
D = 32
DP = 128
H1 = 128
H2 = 64

_W = 8192
_LW = 13
_TG = -(-V1 // (4 * _W))
_VP4 = _TG * _W

_NC = 2
_NS = 16
_NW = _NC * _NS
_BPW = B // _NW

_mesh = plsc.VectorSubcoreMesh(core_axis_name="c", subcore_axis_name="s")

def _tpad_body(t_ref, o_ref):
    x = t_ref[...]
    xs = jnp.concatenate([x[:, a * _W:(a + 1) * _W] for a in range(4)],
                         axis=0)
    eye = jnp.eye(DP, dtype=jnp.float32)
    o_ref[...] = lax.dot_general(xs, eye, (((0,), (0,)), ((), ())),
                                 preferred_element_type=jnp.float32)


def _tpad(tt):
    return pl.pallas_call(
        _tpad_body,
        grid=(_TG,),
        in_specs=[pl.BlockSpec((D, 4 * _W), lambda c: (0, c))],
        out_specs=pl.BlockSpec((_W, DP), lambda c: (c, 0)),
        out_shape=jax.ShapeDtypeStruct((_VP4, DP), jnp.float32),
    )(tt)


@functools.partial(
    pl.kernel,
    mesh=_mesh,
    out_type=jax.ShapeDtypeStruct((B, DP), jnp.float32),
    scratch_types=[
        pltpu.VMEM((_BPW,), jnp.int32),
        pltpu.VMEM((_BPW, DP), jnp.float32),
        pltpu.SemaphoreType.DMA,
    ],
    compiler_params=pltpu.CompilerParams(use_tc_tiling_on_sc=True),
)
def _sc_gather(idx_hbm, tab_hbm, out_hbm, idx_v, rows_v, sem):
    wid = lax.axis_index("s") * _NC + lax.axis_index("c")
    base = wid * _BPW
    pltpu.sync_copy(idx_hbm.at[pl.ds(base, _BPW)], idx_v)
    pltpu.async_copy(tab_hbm.at[idx_v], rows_v, sem).wait()
    pltpu.sync_copy(rows_v, out_hbm.at[pl.ds(base, _BPW)])


def _towers_body(xu_ref, au_ref, xi_ref, ai_ref,
                 uW1_ref, ub1_ref, uW2_ref, ub2_ref, uW3_ref, ub3_ref,
                 iW1_ref, ib1_ref, iW2_ref, ib2_ref, iW3_ref, ib3_ref,
                 uo_ref, io_ref):
    lane_group = lax.broadcasted_iota(jnp.int32, (1, DP), 1) // D

    def select(x128, a):
        return x128 * (lane_group == a).astype(jnp.float32)

    def tower(x, W1, b1, W2, b2, W3, b3):
        h = jnp.maximum(jnp.dot(x, W1, preferred_element_type=jnp.float32) + b1, 0.0)
        h = jnp.maximum(jnp.dot(h, W2, preferred_element_type=jnp.float32) + b2, 0.0)
        return jnp.dot(h, W3, preferred_element_type=jnp.float32) + b3

    xu = select(xu_ref[...], au_ref[...])
    xi = select(xi_ref[...], ai_ref[...])
    uo_ref[...] = tower(xu, uW1_ref[...], ub1_ref[...], uW2_ref[...],
                        ub2_ref[...], uW3_ref[...], ub3_ref[...])
    io_ref[...] = tower(xi, iW1_ref[...], ib1_ref[...], iW2_ref[...],
                        ib2_ref[...], iW3_ref[...], ib3_ref[...])


_BLK = 2048


def _towers(xu, au, xi, ai, weights):
    row_spec = pl.BlockSpec((_BLK, DP), lambda i: (i, 0))
    a_spec = pl.BlockSpec((_BLK, 1), lambda i: (i, 0))
    out_spec = pl.BlockSpec((_BLK, D), lambda i: (i, 0))
    full = lambda shape: pl.BlockSpec(shape, lambda i: (0,) * len(shape))
    w_specs = [
        full((DP, H1)), full((1, H1)), full((H1, H2)), full((1, H2)),
        full((H2, D)), full((1, D)),
        full((DP, H1)), full((1, H1)), full((H1, H2)), full((1, H2)),
        full((H2, D)), full((1, D)),
    ]
    return pl.pallas_call(
        _towers_body,
        grid=(B // _BLK,),
        in_specs=[row_spec, a_spec, row_spec, a_spec] + w_specs,
        out_specs=[out_spec, out_spec],
        out_shape=[
            jax.ShapeDtypeStruct((B, D), jnp.float32),
            jax.ShapeDtypeStruct((B, D), jnp.float32),
        ],
    )(xu, au, xi, ai, *weights)


def kernel(user_id, movie_id, user_table, item_table,
           uW1, ub1, uW2, ub2, uW3, ub3,
           iW1, ib1, iW2, ib2, iW3, ib3):
    uid = user_id.astype(jnp.int32)
    mid = movie_id.astype(jnp.int32)
    utab = _tpad(user_table.T)
    itab = _tpad(item_table.T)

    def packed_row(v):
        return jnp.bitwise_or(
            jnp.left_shift(jnp.right_shift(v, _LW + 2), _LW),
            jnp.bitwise_and(v, _W - 1))

    xu = _sc_gather(packed_row(uid), utab)
    xi = _sc_gather(packed_row(mid), itab)
    au = jnp.bitwise_and(jnp.right_shift(uid, _LW), 3).reshape(B, 1)
    ai = jnp.bitwise_and(jnp.right_shift(mid, _LW), 3).reshape(B, 1)
    weights = (jnp.tile(uW1, (4, 1)), ub1.reshape(1, H1), uW2,
               ub2.reshape(1, H2), uW3, ub3.reshape(1, D),
               jnp.tile(iW1, (4, 1)), ib1.reshape(1, H1), iW2,
               ib2.reshape(1, H2), iW3, ib3.reshape(1, D))
    return _towers(xu, au, xi, ai, weights)


# speedup vs baseline: 2.8308x; 1.0057x over previous
import functools

import jax
import jax.numpy as jnp
from jax import lax
from jax.experimental import pallas as pl
from jax.experimental.pallas import tpu as pltpu
from jax.experimental.pallas import tpu_sc as plsc

B = 16384
V1 = 1000001
D = 32
DP = 128
H1 = 128
H2 = 64

_W = 16384
_LW = 14
_TG = -(-V1 // (4 * _W))
_VP4 = _TG * _W

_NC = 2
_NS = 16
_NW = _NC * _NS
_BPW = B // _NW

_mesh = plsc.VectorSubcoreMesh(core_axis_name="c", subcore_axis_name="s")

def _tpad_body(t_ref, o_ref):
    x = t_ref[...]
    xs = jnp.concatenate([x[:, a * _W:(a + 1) * _W] for a in range(4)],
                         axis=0)
    eye = jnp.eye(DP, dtype=jnp.float32)
    o_ref[...] = lax.dot_general(xs, eye, (((0,), (0,)), ((), ())),
                                 preferred_element_type=jnp.float32)


def _tpad(tt):
    return pl.pallas_call(
        _tpad_body,
        grid=(_TG,),
        in_specs=[pl.BlockSpec((D, 4 * _W), lambda c: (0, c))],
        out_specs=pl.BlockSpec((_W, DP), lambda c: (c, 0)),
        out_shape=jax.ShapeDtypeStruct((_VP4, DP), jnp.float32),
    )(tt)


@functools.partial(
    pl.kernel,
    mesh=_mesh,
    out_type=jax.ShapeDtypeStruct((B, DP), jnp.float32),
    scratch_types=[
        pltpu.VMEM((_BPW,), jnp.int32),
        pltpu.VMEM((_BPW, DP), jnp.float32),
        pltpu.SemaphoreType.DMA,
    ],
    compiler_params=pltpu.CompilerParams(use_tc_tiling_on_sc=True),
)
def _sc_gather(idx_hbm, tab_hbm, out_hbm, idx_v, rows_v, sem):
    wid = lax.axis_index("s") * _NC + lax.axis_index("c")
    base = wid * _BPW
    pltpu.sync_copy(idx_hbm.at[pl.ds(base, _BPW)], idx_v)
    pltpu.async_copy(tab_hbm.at[idx_v], rows_v, sem).wait()
    pltpu.sync_copy(rows_v, out_hbm.at[pl.ds(base, _BPW)])


def _towers_body(xu_ref, au_ref, xi_ref, ai_ref,
                 uW1_ref, ub1_ref, uW2_ref, ub2_ref, uW3_ref, ub3_ref,
                 iW1_ref, ib1_ref, iW2_ref, ib2_ref, iW3_ref, ib3_ref,
                 uo_ref, io_ref):
    lane_group = lax.broadcasted_iota(jnp.int32, (1, DP), 1) // D

    def select(x128, a):
        return x128 * (lane_group == a).astype(jnp.float32)

    def tower(x, W1, b1, W2, b2, W3, b3):
        h = jnp.maximum(jnp.dot(x, W1, preferred_element_type=jnp.float32) + b1, 0.0)
        h = jnp.maximum(jnp.dot(h, W2, preferred_element_type=jnp.float32) + b2, 0.0)
        return jnp.dot(h, W3, preferred_element_type=jnp.float32) + b3

    xu = select(xu_ref[...], au_ref[...])
    xi = select(xi_ref[...], ai_ref[...])
    uo_ref[...] = tower(xu, uW1_ref[...], ub1_ref[...], uW2_ref[...],
                        ub2_ref[...], uW3_ref[...], ub3_ref[...])
    io_ref[...] = tower(xi, iW1_ref[...], ib1_ref[...], iW2_ref[...],
                        ib2_ref[...], iW3_ref[...], ib3_ref[...])


_BLK = 2048


def _towers(xu, au, xi, ai, weights):
    row_spec = pl.BlockSpec((_BLK, DP), lambda i: (i, 0))
    a_spec = pl.BlockSpec((_BLK, 1), lambda i: (i, 0))
    out_spec = pl.BlockSpec((_BLK, D), lambda i: (i, 0))
    full = lambda shape: pl.BlockSpec(shape, lambda i: (0,) * len(shape))
    w_specs = [
        full((DP, H1)), full((1, H1)), full((H1, H2)), full((1, H2)),
        full((H2, D)), full((1, D)),
        full((DP, H1)), full((1, H1)), full((H1, H2)), full((1, H2)),
        full((H2, D)), full((1, D)),
    ]
    return pl.pallas_call(
        _towers_body,
        grid=(B // _BLK,),
        in_specs=[row_spec, a_spec, row_spec, a_spec] + w_specs,
        out_specs=[out_spec, out_spec],
        out_shape=[
            jax.ShapeDtypeStruct((B, D), jnp.float32),
            jax.ShapeDtypeStruct((B, D), jnp.float32),
        ],
    )(xu, au, xi, ai, *weights)


def kernel(user_id, movie_id, user_table, item_table,
           uW1, ub1, uW2, ub2, uW3, ub3,
           iW1, ib1, iW2, ib2, iW3, ib3):
    uid = user_id.astype(jnp.int32)
    mid = movie_id.astype(jnp.int32)
    utab = _tpad(user_table.T)
    itab = _tpad(item_table.T)

    def packed_row(v):
        return jnp.bitwise_or(
            jnp.left_shift(jnp.right_shift(v, _LW + 2), _LW),
            jnp.bitwise_and(v, _W - 1))

    xu = _sc_gather(packed_row(uid), utab)
    xi = _sc_gather(packed_row(mid), itab)
    au = jnp.bitwise_and(jnp.right_shift(uid, _LW), 3).reshape(B, 1)
    ai = jnp.bitwise_and(jnp.right_shift(mid, _LW), 3).reshape(B, 1)
    weights = (jnp.tile(uW1, (4, 1)), ub1.reshape(1, H1), uW2,
               ub2.reshape(1, H2), uW3, ub3.reshape(1, D),
               jnp.tile(iW1, (4, 1)), ib1.reshape(1, H1), iW2,
               ib2.reshape(1, H2), iW3, ib3.reshape(1, D))
    return _towers(xu, au, xi, ai, weights)
